# baseline (speedup 1.0000x reference)
import functools

import jax
import jax.numpy as jnp
from jax import lax
from jax.experimental import pallas as pl
from jax.experimental.pallas import tpu as pltpu
from jax.experimental.pallas import tpu_sc as plsc

D_MODEL = 1024
D_FF = 4096
E = 8
K = 2

BLK = 256
T = 2 * 2048
A = T * K
N_PAD = A + E * BLK
NB = N_PAD // BLK

RB = 512
CH = 512
NCH = T // CH

SC_CORES = 2
SC_SUBCORES = 16
NW = SC_CORES * SC_SUBCORES
TPW = T // NW
CHB = 32
LANES = 16


def _router_body(x_ref, wr_ref, g_ref, i_ref):
    xb = x_ref[...]
    logits = jax.lax.dot_general(
        xb, wr_ref[...], (((1,), (1,)), ((), ())),
        preferred_element_type=jnp.float32)
    e0 = jnp.argmax(logits, axis=-1)
    m0 = jnp.max(logits, axis=-1)
    cols = jax.lax.broadcasted_iota(jnp.int32, logits.shape, 1)
    masked = jnp.where(cols == e0[:, None], -jnp.inf, logits)
    e1 = jnp.argmax(masked, axis=-1)
    m1 = jnp.max(masked, axis=-1)
    g0 = 1.0 / (1.0 + jnp.exp(m1 - m0))
    g_ref[...] = jnp.concatenate([g0[None, :], (1.0 - g0)[None, :]], axis=0)
    i_ref[...] = jnp.concatenate(
        [e0.astype(jnp.int32)[None, :], e1.astype(jnp.int32)[None, :]], axis=0)


def _route(x2d, Wr):
    return pl.pallas_call(
        _router_body,
        grid=(T // RB,),
        in_specs=[
            pl.BlockSpec((RB, D_MODEL), lambda i: (i, 0)),
            pl.BlockSpec((E, D_MODEL), lambda i: (0, 0)),
        ],
        out_specs=[
            pl.BlockSpec((K, RB), lambda i: (0, i)),
            pl.BlockSpec((K, RB), lambda i: (0, i)),
        ],
        out_shape=[
            jax.ShapeDtypeStruct((K, T), jnp.float32),
            jax.ShapeDtypeStruct((K, T), jnp.int32),
        ],
    )(x2d, Wr)


def _meta_body(idx_ref, slot_ref, bmap_ref):
    iota_e = jax.lax.broadcasted_iota(jnp.int32, (E, 1), 0)
    r = jax.lax.broadcasted_iota(jnp.int32, (CH, CH), 0)
    c = jax.lax.broadcasted_iota(jnp.int32, (CH, CH), 1)
    utri_strict = (r < c).astype(jnp.float32)
    dn = (((1,), (0,)), ((), ()))

    def onehot(k, ci):
        e = idx_ref[k:k + 1, pl.ds(ci * CH, CH)]
        return (e == iota_e).astype(jnp.float32)

    counts = jnp.zeros((E, 1), jnp.float32)
    for k in range(K):
        def body(ci, cnt, k=k):
            return cnt + jnp.sum(onehot(k, ci), axis=1, keepdims=True)
        counts = jax.lax.fori_loop(0, NCH, body, counts)

    ltri_e = (jax.lax.broadcasted_iota(jnp.int32, (E, E), 1)
              <= jax.lax.broadcasted_iota(jnp.int32, (E, E), 0)
              ).astype(jnp.float32)
    padded = jnp.ceil(counts / BLK) * BLK
    cum_pad = jax.lax.dot_general(ltri_e, padded, dn,
                                  preferred_element_type=jnp.float32)
    pad_off = cum_pad - padded

    base = pad_off
    for k in range(K):
        def body(ci, b, k=k):
            oh = onehot(k, ci)
            excl = jax.lax.dot_general(oh, utri_strict, dn,
                                       preferred_element_type=jnp.float32)
            slot = jnp.sum(oh * (excl + b), axis=0, keepdims=True)
            slot_ref[k:k + 1, pl.ds(ci * CH, CH)] = slot.astype(jnp.int32)
            return b + jnp.sum(oh, axis=1, keepdims=True)
        base = jax.lax.fori_loop(0, NCH, body, base)

    eye = (jax.lax.broadcasted_iota(jnp.int32, (E, E), 0)
           == jax.lax.broadcasted_iota(jnp.int32, (E, E), 1)
           ).astype(jnp.float32)
    cum_pad_t = jax.lax.dot_general(cum_pad, eye, (((0,), (0,)), ((), ())),
                                    preferred_element_type=jnp.float32)
    total = cum_pad[E - 1, 0]
    starts = (jax.lax.broadcasted_iota(jnp.int32, (NB, 1), 0) * BLK
              ).astype(jnp.float32)
    be = jnp.sum((cum_pad_t <= starts).astype(jnp.int32),
                 axis=1, keepdims=True)
    be = jnp.minimum(be, E - 1)
    dead = starts >= total
    ii = jax.lax.broadcasted_iota(jnp.int32, (NB, 1), 0)
    neg = jnp.full_like(ii, -1)
    be_last = jnp.max(jnp.where(dead, neg, be))
    i_last = jnp.max(jnp.where(dead, neg, ii))
    bmap_ref[...] = jnp.concatenate([
        jnp.where(dead, neg, be),
        jnp.where(dead, be_last, be),
        jnp.where(dead, i_last, ii),
    ], axis=1)


def _metadata(idx):
    return pl.pallas_call(
        _meta_body,
        in_specs=[pl.BlockSpec((K, T), lambda: (0, 0))],
        out_specs=[
            pl.BlockSpec((K, T), lambda: (0, 0)),
            pl.BlockSpec((NB, 3), lambda: (0, 0)),
        ],
        out_shape=[
            jax.ShapeDtypeStruct((K, T), jnp.int32),
            jax.ShapeDtypeStruct((NB, 3), jnp.int32),
        ],
    )(idx)


def _dispatch(x2d, slot_kt):
    @functools.partial(
        pl.kernel,
        mesh=plsc.VectorSubcoreMesh(core_axis_name="c", subcore_axis_name="s"),
        out_type=jax.ShapeDtypeStruct((N_PAD, D_MODEL), jnp.float32),
        scratch_types=[
            pltpu.VMEM((CHB,), jnp.int32),
            pltpu.VMEM((CHB,), jnp.int32),
            pltpu.VMEM((CHB, D_MODEL), jnp.float32),
            pltpu.SemaphoreType.DMA,
            pltpu.SemaphoreType.DMA,
        ],
    )
    def k(x_hbm, slot_hbm, xs_hbm, idx0_v, idx1_v, rows_v, sem0, sem1):
        wid = lax.axis_index("s") * SC_CORES + lax.axis_index("c")
        base = wid * TPW

        def chunk(ci, carry):
            t0 = base + ci * CHB
            pltpu.sync_copy(slot_hbm.at[pl.ds(t0, CHB)], idx0_v)
            pltpu.sync_copy(slot_hbm.at[pl.ds(T + t0, CHB)], idx1_v)
            pltpu.sync_copy(x_hbm.at[pl.ds(t0, CHB), :], rows_v)
            c0 = pltpu.async_copy(rows_v, xs_hbm.at[idx0_v], sem0)
            c1 = pltpu.async_copy(rows_v, xs_hbm.at[idx1_v], sem1)
            c0.wait()
            c1.wait()
            return carry

        lax.fori_loop(0, TPW // CHB, chunk, 0)

    return k(x2d, slot_kt.reshape(K * T))


def _combine(ys, slot_kt, gates_kt):
    @functools.partial(
        pl.kernel,
        mesh=plsc.VectorSubcoreMesh(core_axis_name="c", subcore_axis_name="s"),
        out_type=jax.ShapeDtypeStruct((T, D_MODEL), jnp.float32),
        scratch_types=[
            pltpu.VMEM((CHB,), jnp.int32),
            pltpu.VMEM((CHB,), jnp.int32),
            pltpu.VMEM((CHB,), jnp.float32),
            pltpu.VMEM((CHB,), jnp.float32),
            pltpu.VMEM((CHB, D_MODEL), jnp.float32),
            pltpu.VMEM((CHB, D_MODEL), jnp.float32),
            pltpu.VMEM((CHB, D_MODEL), jnp.float32),
            pltpu.SemaphoreType.DMA,
            pltpu.SemaphoreType.DMA,
        ],
    )
    def k(ys_hbm, slot_hbm, gates_hbm, out_hbm,
          idx0_v, idx1_v, g0_v, g1_v, b0, b1, ob, sem0, sem1):
        wid = lax.axis_index("s") * SC_CORES + lax.axis_index("c")
        base = wid * TPW

        def chunk(ci, carry):
            t0 = base + ci * CHB
            pltpu.sync_copy(slot_hbm.at[pl.ds(t0, CHB)], idx0_v)
            pltpu.sync_copy(slot_hbm.at[pl.ds(T + t0, CHB)], idx1_v)
            c0 = pltpu.async_copy(ys_hbm.at[idx0_v], b0, sem0)
            c1 = pltpu.async_copy(ys_hbm.at[idx1_v], b1, sem1)
            pltpu.sync_copy(gates_hbm.at[pl.ds(t0, CHB)], g0_v)
            pltpu.sync_copy(gates_hbm.at[pl.ds(T + t0, CHB)], g1_v)
            c0.wait()
            c1.wait()

            dnum = lax.GatherDimensionNumbers(
                offset_dims=(), collapsed_slice_dims=(0,),
                start_index_map=(0,))

            def row(rr, c2):
                half = (rr // LANES) * LANES
                lane = jnp.full((LANES, 1), rr - half, jnp.int32)
                g0 = lax.gather(g0_v[pl.ds(half, LANES)], lane, dnum, (1,),
                                mode=lax.GatherScatterMode.PROMISE_IN_BOUNDS)
                g1 = lax.gather(g1_v[pl.ds(half, LANES)], lane, dnum, (1,),
                                mode=lax.GatherScatterMode.PROMISE_IN_BOUNDS)

                def col(dd, c3):
                    sl = pl.ds(dd * LANES, LANES)
                    ob[rr, sl] = g0 * b0[rr, sl] + g1 * b1[rr, sl]
                    return c3

                lax.fori_loop(0, D_MODEL // LANES, col, 0)
                return c2

            lax.fori_loop(0, CHB, row, 0)
            pltpu.sync_copy(ob, out_hbm.at[pl.ds(t0, CHB), :])
            return carry

        lax.fori_loop(0, TPW // CHB, chunk, 0)

    return k(ys, slot_kt.reshape(K * T), gates_kt.reshape(K * T))


NF = 2
FB = D_FF // NF


def _ffn_body(m_ref, xs_ref, wg_ref, wu_ref, wd_ref, yin_ref, ys_ref):
    j = pl.program_id(0)
    i = pl.program_id(1)

    @pl.when(m_ref[i, 0] >= 0)
    def _():
        xsb = xs_ref[...].astype(jnp.bfloat16)
        dn = (((1,), (1,)), ((), ()))
        g = jax.lax.dot_general(xsb, wg_ref[0].astype(jnp.bfloat16), dn,
                                preferred_element_type=jnp.float32)
        u = jax.lax.dot_general(xsb, wu_ref[0].astype(jnp.bfloat16), dn,
                                preferred_element_type=jnp.float32)
        h = g * jax.lax.logistic(g) * u
        y = jax.lax.dot_general(h.astype(jnp.bfloat16),
                                wd_ref[0].astype(jnp.bfloat16), dn,
                                preferred_element_type=jnp.float32)

        @pl.when(j == 0)
        def _():
            ys_ref[...] = y

        @pl.when(j > 0)
        def _():
            ys_ref[...] = yin_ref[...] + y


def _grouped_ffn(xs, Wg, Wu, Wd, bmap):
    grid_spec = pltpu.PrefetchScalarGridSpec(
        num_scalar_prefetch=1,
        grid=(NF, NB),
        in_specs=[
            pl.BlockSpec((BLK, D_MODEL), lambda j, i, m: (m[i, 2], 0)),
            pl.BlockSpec((1, FB, D_MODEL),
                         lambda j, i, m: (m[i, 1], j, 0)),
            pl.BlockSpec((1, FB, D_MODEL),
                         lambda j, i, m: (m[i, 1], j, 0)),
            pl.BlockSpec((1, D_MODEL, FB),
                         lambda j, i, m: (m[i, 1], 0, j)),
            pl.BlockSpec((BLK, D_MODEL),
                         lambda j, i, m: (jnp.where(j == 0, NB - 1, m[i, 2]), 0)),
        ],
        out_specs=pl.BlockSpec((BLK, D_MODEL), lambda j, i, m: (m[i, 2], 0)),
    )
    yin = jnp.zeros((N_PAD, D_MODEL), jnp.float32)
    return pl.pallas_call(
        _ffn_body,
        grid_spec=grid_spec,
        out_shape=jax.ShapeDtypeStruct((N_PAD, D_MODEL), jnp.float32),
        input_output_aliases={5: 0},
    )(bmap, xs, Wg, Wu, Wd, yin)


def kernel(x, Wr, Wg, Wu, Wd):
    B, S, _ = x.shape
    x2d = x.reshape(T, D_MODEL)
    gates_kt, idx_kt = _route(x2d, Wr)
    slot_kt, bmap = _metadata(idx_kt)
    xs = _dispatch(x2d, slot_kt)
    ys = _grouped_ffn(xs, Wg, Wu, Wd, bmap)
    out2d = _combine(ys, slot_kt, gates_kt)
    return out2d.reshape(B, S, D_MODEL)


# speedup vs baseline: 1.0076x; 1.0076x over previous
import functools

import jax
import jax.numpy as jnp
from jax import lax
from jax.experimental import pallas as pl
from jax.experimental.pallas import tpu as pltpu
from jax.experimental.pallas import tpu_sc as plsc

D_MODEL = 1024
D_FF = 4096
E = 8
K = 2

BLK = 256
T = 2 * 2048
A = T * K
N_PAD = A + E * BLK
NB = N_PAD // BLK

RB = 512
CH = 512
NCH = T // CH

SC_CORES = 2
SC_SUBCORES = 16
NW = SC_CORES * SC_SUBCORES
TPW = T // NW
CHB = 32
LANES = 16


def _router_body(x_ref, wr_ref, g_ref, i_ref):
    xb = x_ref[...]
    logits = jax.lax.dot_general(
        xb, wr_ref[...], (((1,), (1,)), ((), ())),
        preferred_element_type=jnp.float32)
    e0 = jnp.argmax(logits, axis=-1)
    m0 = jnp.max(logits, axis=-1)
    cols = jax.lax.broadcasted_iota(jnp.int32, logits.shape, 1)
    masked = jnp.where(cols == e0[:, None], -jnp.inf, logits)
    e1 = jnp.argmax(masked, axis=-1)
    m1 = jnp.max(masked, axis=-1)
    g0 = 1.0 / (1.0 + jnp.exp(m1 - m0))
    g_ref[...] = jnp.concatenate([g0[None, :], (1.0 - g0)[None, :]], axis=0)
    i_ref[...] = jnp.concatenate(
        [e0.astype(jnp.int32)[None, :], e1.astype(jnp.int32)[None, :]], axis=0)


def _route(x2d, Wr):
    return pl.pallas_call(
        _router_body,
        grid=(T // RB,),
        in_specs=[
            pl.BlockSpec((RB, D_MODEL), lambda i: (i, 0)),
            pl.BlockSpec((E, D_MODEL), lambda i: (0, 0)),
        ],
        out_specs=[
            pl.BlockSpec((K, RB), lambda i: (0, i)),
            pl.BlockSpec((K, RB), lambda i: (0, i)),
        ],
        out_shape=[
            jax.ShapeDtypeStruct((K, T), jnp.float32),
            jax.ShapeDtypeStruct((K, T), jnp.int32),
        ],
    )(x2d, Wr)


def _meta_body(idx_ref, slot_ref, bmap_ref):
    iota_e = jax.lax.broadcasted_iota(jnp.int32, (E, 1), 0)
    r = jax.lax.broadcasted_iota(jnp.int32, (CH, CH), 0)
    c = jax.lax.broadcasted_iota(jnp.int32, (CH, CH), 1)
    utri_strict = (r < c).astype(jnp.float32)
    dn = (((1,), (0,)), ((), ()))

    def onehot(k, ci):
        e = idx_ref[k:k + 1, pl.ds(ci * CH, CH)]
        return (e == iota_e).astype(jnp.float32)

    counts = jnp.zeros((E, 1), jnp.float32)
    for k in range(K):
        def body(ci, cnt, k=k):
            return cnt + jnp.sum(onehot(k, ci), axis=1, keepdims=True)
        counts = jax.lax.fori_loop(0, NCH, body, counts)

    ltri_e = (jax.lax.broadcasted_iota(jnp.int32, (E, E), 1)
              <= jax.lax.broadcasted_iota(jnp.int32, (E, E), 0)
              ).astype(jnp.float32)
    padded = jnp.ceil(counts / BLK) * BLK
    cum_pad = jax.lax.dot_general(ltri_e, padded, dn,
                                  preferred_element_type=jnp.float32)
    pad_off = cum_pad - padded

    base = pad_off
    for k in range(K):
        def body(ci, b, k=k):
            oh = onehot(k, ci)
            excl = jax.lax.dot_general(oh, utri_strict, dn,
                                       preferred_element_type=jnp.float32)
            slot = jnp.sum(oh * (excl + b), axis=0, keepdims=True)
            slot_ref[k:k + 1, pl.ds(ci * CH, CH)] = slot.astype(jnp.int32)
            return b + jnp.sum(oh, axis=1, keepdims=True)
        base = jax.lax.fori_loop(0, NCH, body, base)

    eye = (jax.lax.broadcasted_iota(jnp.int32, (E, E), 0)
           == jax.lax.broadcasted_iota(jnp.int32, (E, E), 1)
           ).astype(jnp.float32)
    cum_pad_t = jax.lax.dot_general(cum_pad, eye, (((0,), (0,)), ((), ())),
                                    preferred_element_type=jnp.float32)
    total = cum_pad[E - 1, 0]
    starts = (jax.lax.broadcasted_iota(jnp.int32, (NB, 1), 0) * BLK
              ).astype(jnp.float32)
    be = jnp.sum((cum_pad_t <= starts).astype(jnp.int32),
                 axis=1, keepdims=True)
    be = jnp.minimum(be, E - 1)
    dead = starts >= total
    ii = jax.lax.broadcasted_iota(jnp.int32, (NB, 1), 0)
    neg = jnp.full_like(ii, -1)
    be_last = jnp.max(jnp.where(dead, neg, be))
    i_last = jnp.max(jnp.where(dead, neg, ii))
    bmap_ref[...] = jnp.concatenate([
        jnp.where(dead, neg, be),
        jnp.where(dead, be_last, be),
        jnp.where(dead, i_last, ii),
    ], axis=1)


def _metadata(idx):
    return pl.pallas_call(
        _meta_body,
        in_specs=[pl.BlockSpec((K, T), lambda: (0, 0))],
        out_specs=[
            pl.BlockSpec((K, T), lambda: (0, 0)),
            pl.BlockSpec((NB, 3), lambda: (0, 0)),
        ],
        out_shape=[
            jax.ShapeDtypeStruct((K, T), jnp.int32),
            jax.ShapeDtypeStruct((NB, 3), jnp.int32),
        ],
    )(idx)


def _dispatch(x2d, slot_kt):
    @functools.partial(
        pl.kernel,
        mesh=plsc.VectorSubcoreMesh(core_axis_name="c", subcore_axis_name="s"),
        out_type=jax.ShapeDtypeStruct((N_PAD, D_MODEL), jnp.float32),
        scratch_types=[
            pltpu.VMEM((CHB,), jnp.int32),
            pltpu.VMEM((CHB,), jnp.int32),
            pltpu.VMEM((CHB, D_MODEL), jnp.float32),
            pltpu.SemaphoreType.DMA,
            pltpu.SemaphoreType.DMA,
        ],
    )
    def k(x_hbm, slot_hbm, xs_hbm, idx0_v, idx1_v, rows_v, sem0, sem1):
        wid = lax.axis_index("s") * SC_CORES + lax.axis_index("c")
        base = wid * TPW

        def chunk(ci, carry):
            t0 = base + ci * CHB
            pltpu.sync_copy(slot_hbm.at[pl.ds(t0, CHB)], idx0_v)
            pltpu.sync_copy(slot_hbm.at[pl.ds(T + t0, CHB)], idx1_v)
            pltpu.sync_copy(x_hbm.at[pl.ds(t0, CHB), :], rows_v)
            c0 = pltpu.async_copy(rows_v, xs_hbm.at[idx0_v], sem0)
            c1 = pltpu.async_copy(rows_v, xs_hbm.at[idx1_v], sem1)
            c0.wait()
            c1.wait()
            return carry

        lax.fori_loop(0, TPW // CHB, chunk, 0)

    return k(x2d, slot_kt.reshape(K * T))


def _combine(ys, slot_kt, gates_kt):
    @functools.partial(
        pl.kernel,
        mesh=plsc.VectorSubcoreMesh(core_axis_name="c", subcore_axis_name="s"),
        out_type=jax.ShapeDtypeStruct((T, D_MODEL), jnp.float32),
        scratch_types=[
            pltpu.VMEM((CHB,), jnp.int32),
            pltpu.VMEM((CHB,), jnp.int32),
            pltpu.VMEM((CHB,), jnp.float32),
            pltpu.VMEM((CHB,), jnp.float32),
            pltpu.VMEM((CHB, D_MODEL), jnp.float32),
            pltpu.VMEM((CHB, D_MODEL), jnp.float32),
            pltpu.VMEM((CHB, D_MODEL), jnp.float32),
            pltpu.SemaphoreType.DMA,
            pltpu.SemaphoreType.DMA,
        ],
    )
    def k(ys_hbm, slot_hbm, gates_hbm, out_hbm,
          idx0_v, idx1_v, g0_v, g1_v, b0, b1, ob, sem0, sem1):
        wid = lax.axis_index("s") * SC_CORES + lax.axis_index("c")
        base = wid * TPW

        def chunk(ci, carry):
            t0 = base + ci * CHB
            pltpu.sync_copy(slot_hbm.at[pl.ds(t0, CHB)], idx0_v)
            pltpu.sync_copy(slot_hbm.at[pl.ds(T + t0, CHB)], idx1_v)
            c0 = pltpu.async_copy(ys_hbm.at[idx0_v], b0, sem0)
            c1 = pltpu.async_copy(ys_hbm.at[idx1_v], b1, sem1)
            pltpu.sync_copy(gates_hbm.at[pl.ds(t0, CHB)], g0_v)
            pltpu.sync_copy(gates_hbm.at[pl.ds(T + t0, CHB)], g1_v)
            c0.wait()
            c1.wait()

            dnum = lax.GatherDimensionNumbers(
                offset_dims=(), collapsed_slice_dims=(0,),
                start_index_map=(0,))

            def row(rr, c2):
                half = (rr // LANES) * LANES
                lane = jnp.full((LANES, 1), rr - half, jnp.int32)
                g0 = lax.gather(g0_v[pl.ds(half, LANES)], lane, dnum, (1,),
                                mode=lax.GatherScatterMode.PROMISE_IN_BOUNDS)
                g1 = lax.gather(g1_v[pl.ds(half, LANES)], lane, dnum, (1,),
                                mode=lax.GatherScatterMode.PROMISE_IN_BOUNDS)

                def col(dd, c3):
                    sl = pl.ds(dd * LANES, LANES)
                    ob[rr, sl] = g0 * b0[rr, sl] + g1 * b1[rr, sl]
                    return c3

                lax.fori_loop(0, D_MODEL // LANES, col, 0)
                return c2

            lax.fori_loop(0, CHB, row, 0)
            pltpu.sync_copy(ob, out_hbm.at[pl.ds(t0, CHB), :])
            return carry

        lax.fori_loop(0, TPW // CHB, chunk, 0)

    return k(ys, slot_kt.reshape(K * T), gates_kt.reshape(K * T))


NF = 2
FB = D_FF // NF


def _ffn_body(m_ref, xs_ref, wg_ref, wu_ref, wd_ref, yin_ref, ys_ref):
    j = pl.program_id(0)
    i = pl.program_id(1)

    @pl.when(m_ref[i, 0] >= 0)
    def _():
        xsb = xs_ref[...]
        dn = (((1,), (1,)), ((), ()))
        g = jax.lax.dot_general(xsb, wg_ref[0], dn,
                                preferred_element_type=jnp.float32)
        u = jax.lax.dot_general(xsb, wu_ref[0], dn,
                                preferred_element_type=jnp.float32)
        h = g * jax.lax.logistic(g) * u
        y = jax.lax.dot_general(h, wd_ref[0], dn,
                                preferred_element_type=jnp.float32)

        @pl.when(j == 0)
        def _():
            ys_ref[...] = y

        @pl.when(j > 0)
        def _():
            ys_ref[...] = yin_ref[...] + y


def _grouped_ffn(xs, Wg, Wu, Wd, bmap):
    grid_spec = pltpu.PrefetchScalarGridSpec(
        num_scalar_prefetch=1,
        grid=(NF, NB),
        in_specs=[
            pl.BlockSpec((BLK, D_MODEL), lambda j, i, m: (m[i, 2], 0)),
            pl.BlockSpec((1, FB, D_MODEL),
                         lambda j, i, m: (m[i, 1], j, 0)),
            pl.BlockSpec((1, FB, D_MODEL),
                         lambda j, i, m: (m[i, 1], j, 0)),
            pl.BlockSpec((1, D_MODEL, FB),
                         lambda j, i, m: (m[i, 1], 0, j)),
            pl.BlockSpec((BLK, D_MODEL),
                         lambda j, i, m: (jnp.where(j == 0, NB - 1, m[i, 2]), 0)),
        ],
        out_specs=pl.BlockSpec((BLK, D_MODEL), lambda j, i, m: (m[i, 2], 0)),
    )
    yin = jnp.zeros((N_PAD, D_MODEL), jnp.float32)
    return pl.pallas_call(
        _ffn_body,
        grid_spec=grid_spec,
        out_shape=jax.ShapeDtypeStruct((N_PAD, D_MODEL), jnp.float32),
        input_output_aliases={5: 0},
    )(bmap, xs, Wg, Wu, Wd, yin)


def kernel(x, Wr, Wg, Wu, Wd):
    B, S, _ = x.shape
    x2d = x.reshape(T, D_MODEL)
    gates_kt, idx_kt = _route(x2d, Wr)
    slot_kt, bmap = _metadata(idx_kt)
    xs = _dispatch(x2d, slot_kt)
    ys = _grouped_ffn(xs, Wg, Wu, Wd, bmap)
    out2d = _combine(ys, slot_kt, gates_kt)
    return out2d.reshape(B, S, D_MODEL)
